# baseline (speedup 1.0000x reference)
import jax
import jax.numpy as jnp
from jax.experimental import pallas as pl
from jax.experimental.pallas import tpu as pltpu
from jax.experimental.pallas import tpu_sc as plsc

B, C, H, W = 8, 96, 224, 224
HW = H * W
N = B * HW
EPS = 1e-4
CHUNK = 6272
NCHUNK = HW // CHUNK


def _stats_body(x_ref, s1_ref, s2_ref):
    xb = x_ref[0]
    ps = jnp.sum(xb, axis=1, keepdims=True)
    pq = jnp.sum(xb * xb, axis=1, keepdims=True)
    first = (pl.program_id(0) == 0) & (pl.program_id(1) == 0)

    @pl.when(first)
    def _():
        s1_ref[...] = ps
        s2_ref[...] = pq

    @pl.when(jnp.logical_not(first))
    def _():
        s1_ref[...] += ps
        s2_ref[...] += pq


def _apply_body(x_ref, s1_ref, s2_ref, g0_ref, g1_ref, o_ref):
    inv_n = jnp.float32(1.0 / N)
    mean = s1_ref[...] * inv_n
    var = s2_ref[...] * inv_n - mean * mean
    inv = jax.lax.rsqrt(var + EPS)
    a = inv * (1.0 + g0_ref[0])
    c = g1_ref[0] - mean * a
    o_ref[...] = x_ref[...] * a[None] + c[None]


def _sc_gather(y2, table0, table1):
    mesh = plsc.VectorSubcoreMesh(core_axis_name="c", subcore_axis_name="s")
    cp = table0.shape[1]
    out_t = jax.ShapeDtypeStruct((B, cp), table0.dtype)

    @pl.kernel(out_type=(out_t, out_t), mesh=mesh)
    def k(t0_hbm, t1_hbm, y_hbm, o0_hbm, o1_hbm):
        def body(i_vmem, o0_vmem, o1_vmem):
            pltpu.sync_copy(t0_hbm.at[i_vmem.at[0]], o0_vmem)
            pltpu.sync_copy(t1_hbm.at[i_vmem.at[0]], o1_vmem)

        pltpu.emit_pipeline(
            body,
            grid=(1,),
            in_specs=[pl.BlockSpec((1, B), lambda i: (0, 0))],
            out_specs=[pl.BlockSpec((B, cp), lambda i: (0, 0)),
                       pl.BlockSpec((B, cp), lambda i: (0, 0))],
            core_axis_name="s",
            dimension_semantics=(pltpu.PARALLEL,),
        )(y_hbm, o0_hbm, o1_hbm)

    return k(table0, table1, y2)


def kernel(x, y, embed0, embed1):
    xv = x.reshape(B, C, HW)
    pad = ((0, 0), (0, 128 - C))
    e0y, e1y = _sc_gather(y.reshape(1, B),
                          jnp.pad(embed0, pad), jnp.pad(embed1, pad))
    g0 = e0y[:, :C].reshape(B, C, 1)
    g1 = e1y[:, :C].reshape(B, C, 1)

    s1, s2 = pl.pallas_call(
        _stats_body,
        grid=(B, NCHUNK),
        in_specs=[pl.BlockSpec((1, C, CHUNK), lambda b, j: (b, 0, j))],
        out_specs=[pl.BlockSpec((C, 1), lambda b, j: (0, 0)),
                   pl.BlockSpec((C, 1), lambda b, j: (0, 0))],
        out_shape=[jax.ShapeDtypeStruct((C, 1), jnp.float32)] * 2,
        compiler_params=pltpu.CompilerParams(
            dimension_semantics=("arbitrary", "arbitrary")),
    )(xv)

    out = pl.pallas_call(
        _apply_body,
        grid=(B, NCHUNK),
        in_specs=[pl.BlockSpec((1, C, CHUNK), lambda b, j: (b, 0, j)),
                  pl.BlockSpec((C, 1), lambda b, j: (0, 0)),
                  pl.BlockSpec((C, 1), lambda b, j: (0, 0)),
                  pl.BlockSpec((1, C, 1), lambda b, j: (b, 0, 0)),
                  pl.BlockSpec((1, C, 1), lambda b, j: (b, 0, 0))],
        out_specs=pl.BlockSpec((1, C, CHUNK), lambda b, j: (b, 0, j)),
        out_shape=jax.ShapeDtypeStruct((B, C, HW), jnp.float32),
        compiler_params=pltpu.CompilerParams(
            dimension_semantics=("arbitrary", "arbitrary")),
    )(xv, s1, s2, g0, g1)
    return out.reshape(B, C, H, W)
